# baseline (speedup 1.0000x reference)
import jax
import jax.numpy as jnp
from jax import lax
from jax.experimental import pallas as pl
from jax.experimental.pallas import tpu as pltpu

_H = 8
_C = 4


def _push_kernel(xv_ref, s_ref, o_ref):
    s = s_ref[...]
    rolled = pltpu.roll(s, 1, axis=2)
    a1 = jnp.repeat(xv_ref[...], 8, axis=1)
    s_i = lax.broadcasted_iota(jnp.int32, s.shape, 1)
    l_i = lax.broadcasted_iota(jnp.int32, s.shape, 2)
    idx = 16 * (s_i % 8) + l_i // _H
    xr = jnp.take_along_axis(a1, idx, axis=2)
    o_ref[...] = jnp.where(l_i % _H == 0, xr, rolled)


def _push(xc, sc_):
    Bc, N, H = sc_.shape
    R = N * H // 128
    sv = sc_.reshape(Bc, R, 128)
    xv = xc.reshape(Bc, N // 128, 128)
    bb, sb = 16, 64
    grid = (Bc // bb, R // sb)
    out = pl.pallas_call(
        _push_kernel,
        grid=grid,
        in_specs=[
            pl.BlockSpec((bb, sb // 8, 128), lambda i, j: (i, j, 0)),
            pl.BlockSpec((bb, sb, 128), lambda i, j: (i, j, 0)),
        ],
        out_specs=pl.BlockSpec((bb, sb, 128), lambda i, j: (i, j, 0)),
        out_shape=jax.ShapeDtypeStruct((Bc, R, 128), sc_.dtype),
        compiler_params=pltpu.CompilerParams(
            dimension_semantics=("parallel", "parallel"),
        ),
    )(xv, sv)
    return out.reshape(Bc, N, H)


def kernel(x, state):
    B = state.shape[0]
    Bc = B // _C
    outs = []
    for c in range(_C):
        sc_ = lax.slice_in_dim(state, c * Bc, (c + 1) * Bc, axis=0)
        xc = lax.slice_in_dim(x, c * Bc, (c + 1) * Bc, axis=0)
        outs.append(_push(xc, sc_))
    return jnp.concatenate(outs, axis=0)


# speedup vs baseline: 1.4342x; 1.4342x over previous
import jax
import jax.numpy as jnp
from jax import lax
from jax.experimental import pallas as pl
from jax.experimental.pallas import tpu as pltpu

_H = 8


def _push_kernel(xv_ref, s_ref, o_ref):
    s = s_ref[...]
    rolled = pltpu.roll(s, 1, axis=2)
    a1 = jnp.repeat(xv_ref[...], 8, axis=1)
    s_i = lax.broadcasted_iota(jnp.int32, s.shape, 1)
    l_i = lax.broadcasted_iota(jnp.int32, s.shape, 2)
    idx = 16 * (s_i % 8) + l_i // _H
    xr = jnp.take_along_axis(a1, idx, axis=2)
    o_ref[...] = jnp.where(l_i % _H == 0, xr, rolled)


def kernel(x, state):
    B, N, H = state.shape
    R = N * H // 128
    sv = state.reshape(B, R, 128)
    xv = x.reshape(B, N // 128, 128)
    bb, sb = 16, 128
    grid = (B // bb, R // sb)
    out = pl.pallas_call(
        _push_kernel,
        grid=grid,
        in_specs=[
            pl.BlockSpec((bb, sb // 8, 128), lambda i, j: (i, j, 0)),
            pl.BlockSpec((bb, sb, 128), lambda i, j: (i, j, 0)),
        ],
        out_specs=pl.BlockSpec((bb, sb, 128), lambda i, j: (i, j, 0)),
        out_shape=jax.ShapeDtypeStruct((B, R, 128), state.dtype),
        compiler_params=pltpu.CompilerParams(
            dimension_semantics=("parallel", "parallel"),
        ),
    )(xv, sv)
    return out.reshape(B, N, H)


# speedup vs baseline: 1.5176x; 1.0581x over previous
import jax
import jax.numpy as jnp
from jax import lax
from jax.experimental import pallas as pl
from jax.experimental.pallas import tpu as pltpu

_H = 8


def _push_kernel(xv_ref, s_ref, o_ref):
    s = s_ref[...]
    rolled = pltpu.roll(s, 1, axis=2)
    a1 = jnp.repeat(xv_ref[...], 8, axis=1)
    s_i = lax.broadcasted_iota(jnp.int32, s.shape, 1)
    l_i = lax.broadcasted_iota(jnp.int32, s.shape, 2)
    idx = 16 * (s_i % 8) + l_i // _H
    xr = jnp.take_along_axis(a1, idx, axis=2)
    o_ref[...] = jnp.where(l_i % _H == 0, xr, rolled)


def kernel(x, state):
    B, N, H = state.shape
    R = N * H // 128
    sv = state.reshape(B, R, 128)
    xv = x.reshape(B, N // 128, 128)
    bb, sb = 16, 256
    grid = (B // bb, R // sb)
    out = pl.pallas_call(
        _push_kernel,
        grid=grid,
        in_specs=[
            pl.BlockSpec((bb, sb // 8, 128), lambda i, j: (i, j, 0)),
            pl.BlockSpec((bb, sb, 128), lambda i, j: (i, j, 0)),
        ],
        out_specs=pl.BlockSpec((bb, sb, 128), lambda i, j: (i, j, 0)),
        out_shape=jax.ShapeDtypeStruct((B, R, 128), state.dtype),
        compiler_params=pltpu.CompilerParams(
            dimension_semantics=("parallel", "parallel"),
        ),
    )(xv, sv)
    return out.reshape(B, N, H)


# speedup vs baseline: 1.5607x; 1.0284x over previous
import jax
import jax.numpy as jnp
from jax import lax
from jax.experimental import pallas as pl
from jax.experimental.pallas import tpu as pltpu

_H = 8


def _push_kernel(xv_ref, s_ref, o_ref):
    s = s_ref[...]
    rolled = pltpu.roll(s, 1, axis=2)
    a1 = jnp.repeat(xv_ref[...], 8, axis=1)
    s_i = lax.broadcasted_iota(jnp.int32, s.shape, 1)
    l_i = lax.broadcasted_iota(jnp.int32, s.shape, 2)
    idx = 16 * (s_i % 8) + l_i // _H
    xr = jnp.take_along_axis(a1, idx, axis=2)
    o_ref[...] = jnp.where(l_i % _H == 0, xr, rolled)


def kernel(x, state):
    B, N, H = state.shape
    R = N * H // 128
    sv = state.reshape(B, R, 128)
    xv = x.reshape(B, N // 128, 128)
    bb, sb = 16, 512
    grid = (B // bb, R // sb)
    out = pl.pallas_call(
        _push_kernel,
        grid=grid,
        in_specs=[
            pl.BlockSpec((bb, sb // 8, 128), lambda i, j: (i, j, 0)),
            pl.BlockSpec((bb, sb, 128), lambda i, j: (i, j, 0)),
        ],
        out_specs=pl.BlockSpec((bb, sb, 128), lambda i, j: (i, j, 0)),
        out_shape=jax.ShapeDtypeStruct((B, R, 128), state.dtype),
        compiler_params=pltpu.CompilerParams(
            dimension_semantics=("parallel", "parallel"),
        ),
    )(xv, sv)
    return out.reshape(B, N, H)
